# baseline (speedup 1.0000x reference)
import functools

import jax
import jax.numpy as jnp
from jax import lax
from jax.experimental import pallas as pl
from jax.experimental.pallas import tpu as pltpu
from jax.experimental.pallas import tpu_sc as plsc

_TPS = 2


def kernel(x):
    B, F, K, ED = x.shape
    G = F // _TPS
    NC, NS = 2, 16
    NW = NC * NS
    pairs = B * G
    per_w = pairs // NW

    xt = jnp.swapaxes(x, 2, 3)

    mesh = plsc.VectorSubcoreMesh(core_axis_name="c", subcore_axis_name="s")

    @functools.partial(
        pl.kernel,
        out_type=jax.ShapeDtypeStruct((B, G, _TPS * ED, K), jnp.float32),
        mesh=mesh,
        scratch_types=[pltpu.SemaphoreType.DMA],
    )
    def merge(xt_hbm, outt_hbm, sem):
        wid = lax.axis_index("s") * NC + lax.axis_index("c")
        base = wid * per_w
        copies = []
        for j in range(per_w):
            q = base + j
            b = q // G
            g = q % G
            copies.append(
                pltpu.async_copy(
                    xt_hbm.at[b, _TPS * g],
                    outt_hbm.at[b, g, pl.ds(0, ED), :],
                    sem,
                )
            )
            copies.append(
                pltpu.async_copy(
                    xt_hbm.at[b, _TPS * g + 1],
                    outt_hbm.at[b, g, pl.ds(ED, ED), :],
                    sem,
                )
            )
        for c in copies:
            c.wait()

    outt = merge(xt)
    return jnp.swapaxes(outt, 2, 3)


# speedup vs baseline: 19.9527x; 19.9527x over previous
import functools

import jax
import jax.numpy as jnp
from jax import lax
from jax.experimental import pallas as pl
from jax.experimental.pallas import tpu as pltpu
from jax.experimental.pallas import tpu_sc as plsc

_TPS = 2


def kernel(x):
    B, F, K, ED = x.shape
    G = F // _TPS
    NC, NS = 2, 16
    NW = NC * NS
    pairs = B * G
    per_w = pairs // NW

    xt = jnp.swapaxes(x, 2, 3)

    mesh = plsc.VectorSubcoreMesh(core_axis_name="c", subcore_axis_name="s")

    @functools.partial(
        pl.kernel,
        out_type=jax.ShapeDtypeStruct((B, G, _TPS * ED, K), jnp.float32),
        mesh=mesh,
        scratch_types=[
            pltpu.VMEM((ED, K), jnp.float32),
            pltpu.VMEM((ED, K), jnp.float32),
            pltpu.SemaphoreType.DMA,
            pltpu.SemaphoreType.DMA,
            pltpu.SemaphoreType.DMA,
            pltpu.SemaphoreType.DMA,
        ],
    )
    def merge(xt_hbm, outt_hbm, buf0, buf1, isem0, isem1, osem0, osem1):
        wid = lax.axis_index("s") * NC + lax.axis_index("c")
        base = wid * per_w
        bufs = (buf0, buf1)
        isems = (isem0, isem1)
        osems = (osem0, osem1)
        outs = [None, None]
        for j in range(_TPS * per_w):
            q = base + j // _TPS
            i = j % _TPS
            b = q // G
            g = q % G
            s = j % 2
            if outs[s] is not None:
                outs[s].wait()
            inc = pltpu.async_copy(
                xt_hbm.at[b, _TPS * g + i], bufs[s], isems[s]
            )
            inc.wait()
            outs[s] = pltpu.async_copy(
                bufs[s], outt_hbm.at[b, g, pl.ds(i * ED, ED), :], osems[s]
            )
        for o in outs:
            o.wait()

    outt = merge(xt)
    return jnp.swapaxes(outt, 2, 3)
